# Initial kernel scaffold; baseline (speedup 1.0000x reference)
import jax
import jax.numpy as jnp
from jax.experimental import pallas as pl


def kernel(cache_keys, clip_weights, cache_values, res, value_weights, indices):
    raise NotImplementedError("write your pallas kernel here")


# speedup vs baseline: 1.4504x; 1.4504x over previous
import jax
import jax.numpy as jnp
from jax.experimental import pallas as pl

_FEAT_DIM = 512
_CATE_NUM = 1000
_SHOTS_TOTAL = 32
_FEAT_NUM = 256
_ROWS = _CATE_NUM * _SHOTS_TOTAL

_BLK_ROWS = 256
_BLK_CLS = _BLK_ROWS // _SHOTS_TOTAL


def _body(idx_ref, res_full_ref, cw_ref, ck_ref, cv_ref, vw_ref, res_blk_ref,
          nck_ref, ncv_ref, ncw_ref):
    col = jax.lax.broadcasted_iota(jnp.int32, (_FEAT_NUM, _FEAT_DIM), 1)
    s = (idx_ref[...] == col).astype(jnp.float32)

    add8 = jnp.dot(res_blk_ref[...], s, preferred_element_type=jnp.float32)
    rr = jax.lax.broadcasted_iota(jnp.int32, (_BLK_ROWS, _BLK_CLS), 0) // _SHOTS_TOTAL
    cc = jax.lax.broadcasted_iota(jnp.int32, (_BLK_ROWS, _BLK_CLS), 1)
    rep = (rr == cc).astype(jnp.float32)
    nck_ref[...] = ck_ref[...] + jnp.dot(rep, add8, preferred_element_type=jnp.float32)

    ncv_ref[...] = cv_ref[...] * vw_ref[...]

    @pl.when(pl.program_id(0) == 0)
    def _():
        ncw_ref[...] = cw_ref[...] + jax.lax.dot_general(
            s, res_full_ref[...], (((0,), (1,)), ((), ())),
            preferred_element_type=jnp.float32)


def kernel(cache_keys, clip_weights, cache_values, res, value_weights, indices):
    idx = indices.astype(jnp.int32).reshape(_FEAT_NUM, 1)
    grid = _ROWS // _BLK_ROWS
    out = pl.pallas_call(
        _body,
        grid=(grid,),
        in_specs=[
            pl.BlockSpec((_FEAT_NUM, 1), lambda i: (0, 0)),
            pl.BlockSpec((_CATE_NUM, _FEAT_NUM), lambda i: (0, 0)),
            pl.BlockSpec((_FEAT_DIM, _CATE_NUM), lambda i: (0, 0)),
            pl.BlockSpec((_BLK_ROWS, _FEAT_DIM), lambda i: (i, 0)),
            pl.BlockSpec((_BLK_ROWS, _CATE_NUM), lambda i: (i, 0)),
            pl.BlockSpec((_BLK_ROWS, 1), lambda i: (i, 0)),
            pl.BlockSpec((_BLK_CLS, _FEAT_NUM), lambda i: (i, 0)),
        ],
        out_specs=[
            pl.BlockSpec((_BLK_ROWS, _FEAT_DIM), lambda i: (i, 0)),
            pl.BlockSpec((_BLK_ROWS, _CATE_NUM), lambda i: (i, 0)),
            pl.BlockSpec((_FEAT_DIM, _CATE_NUM), lambda i: (0, 0)),
        ],
        out_shape=[
            jax.ShapeDtypeStruct((_ROWS, _FEAT_DIM), jnp.float32),
            jax.ShapeDtypeStruct((_ROWS, _CATE_NUM), jnp.float32),
            jax.ShapeDtypeStruct((_FEAT_DIM, _CATE_NUM), jnp.float32),
        ],
    )(idx, res, clip_weights, cache_keys, cache_values, value_weights, res)
    return (out[0], out[2], out[1])


# speedup vs baseline: 1.6550x; 1.1411x over previous
import jax
import jax.numpy as jnp
from jax.experimental import pallas as pl
from jax.experimental.pallas import tpu as pltpu

_FEAT_DIM = 512
_CATE_NUM = 1000
_SHOTS_TOTAL = 32
_FEAT_NUM = 256

_BLK_CLS = 8


def _body(idx_ref, res_full_ref, cw_ref, ck_ref, cv_ref, vw_ref,
          nck_ref, ncv_ref, ncw_ref, res_exp_ref):
    i = pl.program_id(0)

    @pl.when(i == 0)
    def _():
        col = jax.lax.broadcasted_iota(jnp.int32, (_FEAT_NUM, _FEAT_DIM), 1)
        s = (idx_ref[...] == col).astype(jnp.float32)
        res_exp_ref[...] = jnp.dot(res_full_ref[...], s,
                                   preferred_element_type=jnp.float32)
        ncw_ref[...] = cw_ref[...] + jax.lax.dot_general(
            s, res_full_ref[...], (((0,), (1,)), ((), ())),
            preferred_element_type=jnp.float32)

    add = res_exp_ref[pl.ds(i * _BLK_CLS, _BLK_CLS), :]
    nck_ref[...] = ck_ref[...] + add[:, None, :]
    ncv_ref[...] = cv_ref[...] * vw_ref[...]


def kernel(cache_keys, clip_weights, cache_values, res, value_weights, indices):
    idx = indices.astype(jnp.int32).reshape(_FEAT_NUM, 1)
    ck3 = cache_keys.reshape(_CATE_NUM, _SHOTS_TOTAL, _FEAT_DIM)
    cv3 = cache_values.reshape(_CATE_NUM, _SHOTS_TOTAL, _CATE_NUM)
    vw3 = value_weights.reshape(_CATE_NUM, _SHOTS_TOTAL, 1)
    grid = _CATE_NUM // _BLK_CLS
    out = pl.pallas_call(
        _body,
        grid=(grid,),
        in_specs=[
            pl.BlockSpec((_FEAT_NUM, 1), lambda i: (0, 0)),
            pl.BlockSpec((_CATE_NUM, _FEAT_NUM), lambda i: (0, 0)),
            pl.BlockSpec((_FEAT_DIM, _CATE_NUM), lambda i: (0, 0)),
            pl.BlockSpec((_BLK_CLS, _SHOTS_TOTAL, _FEAT_DIM), lambda i: (i, 0, 0)),
            pl.BlockSpec((_BLK_CLS, _SHOTS_TOTAL, _CATE_NUM), lambda i: (i, 0, 0)),
            pl.BlockSpec((_BLK_CLS, _SHOTS_TOTAL, 1), lambda i: (i, 0, 0)),
        ],
        out_specs=[
            pl.BlockSpec((_BLK_CLS, _SHOTS_TOTAL, _FEAT_DIM), lambda i: (i, 0, 0)),
            pl.BlockSpec((_BLK_CLS, _SHOTS_TOTAL, _CATE_NUM), lambda i: (i, 0, 0)),
            pl.BlockSpec((_FEAT_DIM, _CATE_NUM), lambda i: (0, 0)),
        ],
        out_shape=[
            jax.ShapeDtypeStruct((_CATE_NUM, _SHOTS_TOTAL, _FEAT_DIM), jnp.float32),
            jax.ShapeDtypeStruct((_CATE_NUM, _SHOTS_TOTAL, _CATE_NUM), jnp.float32),
            jax.ShapeDtypeStruct((_FEAT_DIM, _CATE_NUM), jnp.float32),
        ],
        scratch_shapes=[pltpu.VMEM((_CATE_NUM, _FEAT_DIM), jnp.float32)],
    )(idx, res, clip_weights, ck3, cv3, vw3)
    nck = out[0].reshape(_CATE_NUM * _SHOTS_TOTAL, _FEAT_DIM)
    ncv = out[1].reshape(_CATE_NUM * _SHOTS_TOTAL, _CATE_NUM)
    return (nck, out[2], ncv)
